# initial kernel scaffold (unmeasured)
import jax
import jax.numpy as jnp
from jax import lax
from jax.experimental import pallas as pl
from jax.experimental.pallas import tpu as pltpu

N_DEV = 4
N_LOCAL_E = 8
N_TOK = 2048
D = 1024
BLK = N_TOK // N_DEV


def kernel(x, router_W, route_idx, expert_W, shared_W):
    my = lax.axis_index("i")
    scores = x @ router_W
    scores = scores - scores.max(axis=-1, keepdims=True)
    p = jnp.exp(scores)
    probs = p / p.sum(axis=-1, keepdims=True)
    w = jnp.take_along_axis(probs, route_idx, axis=1)
    local_e = route_idx[:, 0] - my * N_LOCAL_E
    gates = jax.nn.one_hot(local_e, N_LOCAL_E, dtype=jnp.float32) * w

    x_bf = x.astype(jnp.bfloat16)
    ew_bf = expert_W.astype(jnp.bfloat16)
    sw_bf = shared_W.astype(jnp.bfloat16)

    def body(x_ref, g_ref, ew_ref, sw_ref, out_ref,
             comm_ref, send_sems, recv_sems, credit_sem):
        my_pos = lax.axis_index("i")
        left = (my_pos + N_DEV - 1) % N_DEV
        right = (my_pos + 1) % N_DEV

        barrier_sem = pltpu.get_barrier_semaphore()
        for nbr in (left, right):
            pl.semaphore_signal(barrier_sem, inc=1, device_id=(nbr,),
                                device_id_type=pl.DeviceIdType.MESH)
        pl.semaphore_wait(barrier_sem, 2)

        def partial_block(b):
            xb = x_ref[pl.ds(b * BLK, BLK), :]
            gb = g_ref[pl.ds(b * BLK, BLK), :]
            acc = jnp.zeros((BLK, D), jnp.float32)
            for j in range(N_LOCAL_E):
                acc = acc + gb[:, j:j + 1] * jnp.dot(
                    xb, ew_ref[j], preferred_element_type=jnp.float32)
            return acc

        comm_ref[0, :, :] = partial_block((my_pos + N_DEV - 1) % N_DEV
                                          ).astype(jnp.bfloat16)

        for s in range(N_DEV - 1):
            send_slot = s % 2
            recv_slot = (s + 1) % 2
            if s == 2:
                pl.semaphore_wait(credit_sem, 1)
            rdma = pltpu.make_async_remote_copy(
                src_ref=comm_ref.at[send_slot],
                dst_ref=comm_ref.at[recv_slot],
                send_sem=send_sems.at[send_slot],
                recv_sem=recv_sems.at[recv_slot],
                device_id=(right,),
                device_id_type=pl.DeviceIdType.MESH,
            )
            rdma.start()

            b_next = (my_pos + N_DEV - 2 - s) % N_DEV
            p_next = partial_block(b_next)
            if s == N_DEV - 2:
                xb = x_ref[pl.ds(my_pos * BLK, BLK), :]
                p_next = p_next + jnp.dot(
                    xb, sw_ref[:, :], preferred_element_type=jnp.float32)

            rdma.wait()
            if s == 1:
                pl.semaphore_signal(credit_sem, inc=1, device_id=(left,),
                                    device_id_type=pl.DeviceIdType.MESH)
            if s < N_DEV - 2:
                comm_ref[recv_slot, :, :] = (
                    comm_ref[recv_slot, :, :].astype(jnp.float32) + p_next
                ).astype(jnp.bfloat16)
            else:
                out_ref[:, :] = comm_ref[recv_slot, :, :].astype(jnp.float32) + p_next

    return pl.pallas_call(
        body,
        out_shape=jax.ShapeDtypeStruct((BLK, D), jnp.float32),
        in_specs=[pl.BlockSpec(memory_space=pltpu.VMEM)] * 4,
        out_specs=pl.BlockSpec(memory_space=pltpu.VMEM),
        scratch_shapes=[
            pltpu.VMEM((2, BLK, D), jnp.bfloat16),
            pltpu.SemaphoreType.DMA((2,)),
            pltpu.SemaphoreType.DMA((2,)),
            pltpu.SemaphoreType.REGULAR,
        ],
        compiler_params=pltpu.CompilerParams(collective_id=0),
    )(x_bf, gates, ew_bf, sw_bf)


# baseline (device time: 81571 ns/iter reference)
import jax
import jax.numpy as jnp
from jax import lax
from jax.experimental import pallas as pl
from jax.experimental.pallas import tpu as pltpu

N_DEV = 4
N_EXPERTS = 32
N_LOCAL_E = 8
N_TOK = 2048
D = 1024
BLK = N_TOK // N_DEV


def kernel(x, router_W, route_idx, expert_W, shared_W):
    def body(x_ref, rw_ref, idx_ref, ew_ref, sw_ref, out_ref,
             xbf_ref, g_ref, ewbf_ref, stage_ref, comm_ref,
             copy_sems, send_sems, recv_sems, credit_sem):
        my_pos = lax.axis_index("i")
        left = (my_pos + N_DEV - 1) % N_DEV
        right = (my_pos + 1) % N_DEV

        def ew_copy(j):
            return pltpu.make_async_copy(
                ew_ref.at[j], stage_ref.at[j % 2], copy_sems.at[j % 2])

        ew_copy(0).start()
        ew_copy(1).start()

        xbf_ref[:, :] = x_ref[:, :].astype(jnp.bfloat16)
        scores = jnp.dot(x_ref[:, :], rw_ref[:, :],
                         preferred_element_type=jnp.float32)
        scores = scores - jnp.max(scores, axis=-1, keepdims=True)
        p = jnp.exp(scores)
        probs = p / jnp.sum(p, axis=-1, keepdims=True)
        e = idx_ref[:, :]
        iota32 = lax.broadcasted_iota(jnp.int32, (N_TOK, N_EXPERTS), 1)
        w = jnp.sum(jnp.where(iota32 == e, probs, 0.0),
                    axis=-1, keepdims=True)
        iota8 = lax.broadcasted_iota(jnp.int32, (N_TOK, N_LOCAL_E), 1)
        g_ref[:, :] = jnp.where(iota8 == e - my_pos * N_LOCAL_E, w, 0.0)

        for j in range(N_LOCAL_E):
            ew_copy(j).wait()
            if j + 2 < N_LOCAL_E:
                ew_copy(j + 2).start()
            ewbf_ref[j, :, :] = stage_ref[j % 2, :, :].astype(jnp.bfloat16)

        barrier_sem = pltpu.get_barrier_semaphore()
        for nbr in (left, right):
            pl.semaphore_signal(barrier_sem, inc=1, device_id=(nbr,),
                                device_id_type=pl.DeviceIdType.MESH)
        pl.semaphore_wait(barrier_sem, 2)

        def partial_block(b):
            xb = xbf_ref[pl.ds(b * BLK, BLK), :]
            gb = g_ref[pl.ds(b * BLK, BLK), :]
            acc = jnp.zeros((BLK, D), jnp.float32)
            for j in range(N_LOCAL_E):
                acc = acc + gb[:, j:j + 1] * jnp.dot(
                    xb, ewbf_ref[j],
                    preferred_element_type=jnp.float32)
            return acc

        comm_ref[0, :, :] = partial_block((my_pos + N_DEV - 1) % N_DEV
                                          ).astype(jnp.bfloat16)

        for s in range(N_DEV - 1):
            send_slot = s % 2
            recv_slot = (s + 1) % 2
            if s == 2:
                pl.semaphore_wait(credit_sem, 1)
            rdma = pltpu.make_async_remote_copy(
                src_ref=comm_ref.at[send_slot],
                dst_ref=comm_ref.at[recv_slot],
                send_sem=send_sems.at[send_slot],
                recv_sem=recv_sems.at[recv_slot],
                device_id=(right,),
                device_id_type=pl.DeviceIdType.MESH,
            )
            rdma.start()

            b_next = (my_pos + N_DEV - 2 - s) % N_DEV
            p_next = partial_block(b_next)
            if s == N_DEV - 2:
                xb = xbf_ref[pl.ds(my_pos * BLK, BLK), :]
                p_next = p_next + jnp.dot(
                    xb, sw_ref[:, :].astype(jnp.bfloat16),
                    preferred_element_type=jnp.float32)

            rdma.wait()
            if s == 1:
                pl.semaphore_signal(credit_sem, inc=1, device_id=(left,),
                                    device_id_type=pl.DeviceIdType.MESH)
            if s < N_DEV - 2:
                comm_ref[recv_slot, :, :] = (
                    comm_ref[recv_slot, :, :].astype(jnp.float32) + p_next
                ).astype(jnp.bfloat16)
            else:
                out_ref[:, :] = comm_ref[recv_slot, :, :].astype(jnp.float32) + p_next

    return pl.pallas_call(
        body,
        out_shape=jax.ShapeDtypeStruct((BLK, D), jnp.float32),
        in_specs=[
            pl.BlockSpec(memory_space=pltpu.VMEM),
            pl.BlockSpec(memory_space=pltpu.VMEM),
            pl.BlockSpec(memory_space=pltpu.VMEM),
            pl.BlockSpec(memory_space=pl.ANY),
            pl.BlockSpec(memory_space=pltpu.VMEM),
        ],
        out_specs=pl.BlockSpec(memory_space=pltpu.VMEM),
        scratch_shapes=[
            pltpu.VMEM((N_TOK, D), jnp.bfloat16),
            pltpu.VMEM((N_TOK, N_LOCAL_E), jnp.float32),
            pltpu.VMEM((N_LOCAL_E, D, D), jnp.bfloat16),
            pltpu.VMEM((2, D, D), jnp.float32),
            pltpu.VMEM((2, BLK, D), jnp.bfloat16),
            pltpu.SemaphoreType.DMA((2,)),
            pltpu.SemaphoreType.DMA((2,)),
            pltpu.SemaphoreType.DMA((2,)),
            pltpu.SemaphoreType.REGULAR,
        ],
        compiler_params=pltpu.CompilerParams(
            collective_id=0, vmem_limit_bytes=64 * 1024 * 1024),
    )(x, router_W, route_idx, expert_W, shared_W)


# device time: 70754 ns/iter; 1.1529x vs baseline; 1.1529x over previous
import jax
import jax.numpy as jnp
from jax import lax
from jax.experimental import pallas as pl
from jax.experimental.pallas import tpu as pltpu

N_DEV = 4
N_EXPERTS = 32
N_LOCAL_E = 8
N_TOK = 2048
D = 1024
BLK = N_TOK // N_DEV


def kernel(x, router_W, route_idx, expert_W, shared_W):
    def body(x_ref, rw_ref, idx_ref, ew_ref, sw_ref, out_ref,
             xbf_ref, g_ref, ewbf_ref, stage_ref, send_ref, recv_ref,
             copy_sems, send_sems, recv_sems):
        my_pos = lax.axis_index("i")

        def ew_copy(j):
            return pltpu.make_async_copy(
                ew_ref.at[j], stage_ref.at[j % 2], copy_sems.at[j % 2])

        ew_copy(0).start()
        ew_copy(1).start()

        xbf_ref[:, :] = x_ref[:, :].astype(jnp.bfloat16)
        scores = jnp.dot(x_ref[:, :], rw_ref[:, :],
                         preferred_element_type=jnp.float32)
        scores = scores - jnp.max(scores, axis=-1, keepdims=True)
        p = jnp.exp(scores)
        probs = p / jnp.sum(p, axis=-1, keepdims=True)
        e = idx_ref[:, :]
        iota32 = lax.broadcasted_iota(jnp.int32, (N_TOK, N_EXPERTS), 1)
        w = jnp.sum(jnp.where(iota32 == e, probs, 0.0),
                    axis=-1, keepdims=True)
        iota8 = lax.broadcasted_iota(jnp.int32, (N_TOK, N_LOCAL_E), 1)
        g_ref[:, :] = jnp.where(iota8 == e - my_pos * N_LOCAL_E, w, 0.0)

        barrier_sem = pltpu.get_barrier_semaphore()
        for m in range(1, N_DEV):
            pl.semaphore_signal(barrier_sem, inc=1,
                                device_id=((my_pos + m) % N_DEV,),
                                device_id_type=pl.DeviceIdType.MESH)
        pl.semaphore_wait(barrier_sem, N_DEV - 1)

        def drain(j):
            ew_copy(j).wait()
            ewbf_ref[j, :, :] = stage_ref[j % 2, :, :].astype(jnp.bfloat16)
            if j + 2 < N_LOCAL_E:
                ew_copy(j + 2).start()

        def partial_block(b, j_hook):
            xb = xbf_ref[pl.ds(b * BLK, BLK), :]
            gb = g_ref[pl.ds(b * BLK, BLK), :]
            acc = jnp.zeros((BLK, D), jnp.float32)
            for j in range(N_LOCAL_E):
                j_hook(j)
                acc = acc + gb[:, j:j + 1] * jnp.dot(
                    xb, ewbf_ref[j], preferred_element_type=jnp.float32)
            return acc

        rdmas = []
        for m in range(1, N_DEV):
            b = (my_pos + m) % N_DEV
            part = partial_block(b, drain if m == 1 else (lambda j: None))
            send_ref[m - 1, :, :] = part.astype(jnp.bfloat16)
            rdma = pltpu.make_async_remote_copy(
                src_ref=send_ref.at[m - 1],
                dst_ref=recv_ref.at[m - 1],
                send_sem=send_sems.at[m - 1],
                recv_sem=recv_sems.at[m - 1],
                device_id=(b,),
                device_id_type=pl.DeviceIdType.MESH,
            )
            rdma.start()
            rdmas.append(rdma)

        p_own = partial_block(my_pos, lambda j: None)
        xb = xbf_ref[pl.ds(my_pos * BLK, BLK), :]
        p_own = p_own + jnp.dot(xb, sw_ref[:, :].astype(jnp.bfloat16),
                                preferred_element_type=jnp.float32)

        for rdma in rdmas:
            rdma.wait_recv()
        out_ref[:, :] = (p_own
                         + recv_ref[0, :, :].astype(jnp.float32)
                         + recv_ref[1, :, :].astype(jnp.float32)
                         + recv_ref[2, :, :].astype(jnp.float32))
        for rdma in rdmas:
            rdma.wait_send()

    return pl.pallas_call(
        body,
        out_shape=jax.ShapeDtypeStruct((BLK, D), jnp.float32),
        in_specs=[
            pl.BlockSpec(memory_space=pltpu.VMEM),
            pl.BlockSpec(memory_space=pltpu.VMEM),
            pl.BlockSpec(memory_space=pltpu.VMEM),
            pl.BlockSpec(memory_space=pl.ANY),
            pl.BlockSpec(memory_space=pltpu.VMEM),
        ],
        out_specs=pl.BlockSpec(memory_space=pltpu.VMEM),
        scratch_shapes=[
            pltpu.VMEM((N_TOK, D), jnp.bfloat16),
            pltpu.VMEM((N_TOK, N_LOCAL_E), jnp.float32),
            pltpu.VMEM((N_LOCAL_E, D, D), jnp.bfloat16),
            pltpu.VMEM((2, D, D), jnp.float32),
            pltpu.VMEM((N_DEV - 1, BLK, D), jnp.bfloat16),
            pltpu.VMEM((N_DEV - 1, BLK, D), jnp.bfloat16),
            pltpu.SemaphoreType.DMA((2,)),
            pltpu.SemaphoreType.DMA((N_DEV - 1,)),
            pltpu.SemaphoreType.DMA((N_DEV - 1,)),
        ],
        compiler_params=pltpu.CompilerParams(
            collective_id=0, vmem_limit_bytes=64 * 1024 * 1024),
    )(x, router_W, route_idx, expert_W, shared_W)


# device time: 68034 ns/iter; 1.1990x vs baseline; 1.0400x over previous
import jax
import jax.numpy as jnp
from jax import lax
from jax.experimental import pallas as pl
from jax.experimental.pallas import tpu as pltpu

N_DEV = 4
N_EXPERTS = 32
N_LOCAL_E = 8
N_TOK = 2048
D = 1024
BLK = N_TOK // N_DEV


def kernel(x, router_W, route_idx, expert_W, shared_W):
    def body(x_ref, rw_ref, idx_ref, ew_ref, sw_ref, out_ref,
             xbf_ref, g_ref, ewbf_ref, stage_ref, send_ref, recv_ref,
             copy_sems, send_sems, recv_sems):
        my_pos = lax.axis_index("i")

        N_STAGE = 2

        def ew_copy(j):
            return pltpu.make_async_copy(
                ew_ref.at[j], stage_ref.at[j % N_STAGE], copy_sems.at[j % N_STAGE])

        for j in range(N_STAGE):
            ew_copy(j).start()

        xbf_ref[:, :] = x_ref[:, :].astype(jnp.bfloat16)
        scores = jnp.dot(x_ref[:, :], rw_ref[:, :],
                         preferred_element_type=jnp.float32)
        scores = scores - jnp.max(scores, axis=-1, keepdims=True)
        p = jnp.exp(scores)
        probs = p / jnp.sum(p, axis=-1, keepdims=True)
        e = idx_ref[:, :]
        iota32 = lax.broadcasted_iota(jnp.int32, (N_TOK, N_EXPERTS), 1)
        w = jnp.sum(jnp.where(iota32 == e, probs, 0.0),
                    axis=-1, keepdims=True)
        iota8 = lax.broadcasted_iota(jnp.int32, (N_TOK, N_LOCAL_E), 1)
        g_ref[:, :] = jnp.where(iota8 == e - my_pos * N_LOCAL_E, w, 0.0)

        barrier_sem = pltpu.get_barrier_semaphore()
        for m in range(1, N_DEV):
            pl.semaphore_signal(barrier_sem, inc=1,
                                device_id=((my_pos + m) % N_DEV,),
                                device_id_type=pl.DeviceIdType.MESH)
        pl.semaphore_wait(barrier_sem, N_DEV - 1)

        def drain(j):
            ew_copy(j).wait()
            ewbf_ref[j, :, :] = stage_ref[j % N_STAGE, :, :].astype(jnp.bfloat16)
            if j + N_STAGE < N_LOCAL_E:
                ew_copy(j + N_STAGE).start()

        def block_operands(b):
            xb = xbf_ref[pl.ds(b * BLK, BLK), :]
            gb = g_ref[pl.ds(b * BLK, BLK), :]
            return xb, gb

        def start_send(m, part):
            send_ref[m - 1, :, :] = part.astype(jnp.bfloat16)
            rdma = pltpu.make_async_remote_copy(
                src_ref=send_ref.at[m - 1],
                dst_ref=recv_ref.at[m - 1],
                send_sem=send_sems.at[m - 1],
                recv_sem=recv_sems.at[m - 1],
                device_id=((my_pos + m) % N_DEV,),
                device_id_type=pl.DeviceIdType.MESH,
            )
            rdma.start()
            return rdma

        xb1, gb1 = block_operands((my_pos + 1) % N_DEV)
        xb2, gb2 = block_operands((my_pos + 2) % N_DEV)
        acc1 = jnp.zeros((BLK, D), jnp.float32)
        acc2 = jnp.zeros((BLK, D), jnp.float32)
        for j in range(N_LOCAL_E):
            drain(j)
            acc1 = acc1 + gb1[:, j:j + 1] * jnp.dot(
                xb1, ewbf_ref[j], preferred_element_type=jnp.float32)
            acc2 = acc2 + gb2[:, j:j + 1] * jnp.dot(
                xb2, ewbf_ref[j], preferred_element_type=jnp.float32)
        rdmas = [start_send(1, acc1), start_send(2, acc2)]

        xb3, gb3 = block_operands((my_pos + 3) % N_DEV)
        acc3 = jnp.zeros((BLK, D), jnp.float32)
        for j in range(N_LOCAL_E):
            acc3 = acc3 + gb3[:, j:j + 1] * jnp.dot(
                xb3, ewbf_ref[j], preferred_element_type=jnp.float32)
        rdmas.append(start_send(3, acc3))

        xb0, gb0 = block_operands(my_pos)
        p_own = jnp.dot(xb0, sw_ref[:, :].astype(jnp.bfloat16),
                        preferred_element_type=jnp.float32)
        for j in range(N_LOCAL_E):
            p_own = p_own + gb0[:, j:j + 1] * jnp.dot(
                xb0, ewbf_ref[j], preferred_element_type=jnp.float32)

        for rdma in rdmas:
            rdma.wait_recv()
        out_ref[:, :] = (p_own
                         + recv_ref[0, :, :].astype(jnp.float32)
                         + recv_ref[1, :, :].astype(jnp.float32)
                         + recv_ref[2, :, :].astype(jnp.float32))
        for rdma in rdmas:
            rdma.wait_send()

    return pl.pallas_call(
        body,
        out_shape=jax.ShapeDtypeStruct((BLK, D), jnp.float32),
        in_specs=[
            pl.BlockSpec(memory_space=pltpu.VMEM),
            pl.BlockSpec(memory_space=pltpu.VMEM),
            pl.BlockSpec(memory_space=pltpu.VMEM),
            pl.BlockSpec(memory_space=pl.ANY),
            pl.BlockSpec(memory_space=pltpu.VMEM),
        ],
        out_specs=pl.BlockSpec(memory_space=pltpu.VMEM),
        scratch_shapes=[
            pltpu.VMEM((N_TOK, D), jnp.bfloat16),
            pltpu.VMEM((N_TOK, N_LOCAL_E), jnp.float32),
            pltpu.VMEM((N_LOCAL_E, D, D), jnp.bfloat16),
            pltpu.VMEM((2, D, D), jnp.float32),
            pltpu.VMEM((N_DEV - 1, BLK, D), jnp.bfloat16),
            pltpu.VMEM((N_DEV - 1, BLK, D), jnp.bfloat16),
            pltpu.SemaphoreType.DMA((2,)),
            pltpu.SemaphoreType.DMA((N_DEV - 1,)),
            pltpu.SemaphoreType.DMA((N_DEV - 1,)),
        ],
        compiler_params=pltpu.CompilerParams(
            collective_id=0, vmem_limit_bytes=64 * 1024 * 1024),
    )(x, router_W, route_idx, expert_W, shared_W)
